# baseline (speedup 1.0000x reference)
import jax
import jax.numpy as jnp
from jax import lax
from jax.experimental import pallas as pl
from jax.experimental.pallas import tpu as pltpu

_R, _C = 16384, 2048
_BLK = 1024
_TCH = 8
_TUN = 128


def _loss_chunk(x, lab):
    eps = jnp.float32(1e-31)
    y = x * x + 1.0
    s = y * lax.rsqrt(y)
    t = jnp.exp(1.0 - s)
    w = jnp.where(lab == 1, jnp.maximum(1.0 - t, eps), t)
    return jnp.log(w)


def _tc_body(out_ref, lab_ref, sum_ref):
    def step(j, acc):
        x = out_ref[pl.ds(j * _TCH, _TCH), :]
        lab = lab_ref[pl.ds(j * _TCH, _TCH), :]
        return acc + _loss_chunk(x, lab)

    acc = lax.fori_loop(
        0, _BLK // _TCH, step, jnp.zeros((_TCH, _C), jnp.float32),
        unroll=_TUN,
    )
    part = jnp.sum(acc)

    @pl.when(pl.program_id(0) == 0)
    def _():
        sum_ref[0, 0] = 0.0

    sum_ref[0, 0] -= part


def kernel(out, label):
    total = pl.pallas_call(
        _tc_body,
        grid=(_R // _BLK,),
        in_specs=[
            pl.BlockSpec((_BLK, _C), lambda i: (i, 0)),
            pl.BlockSpec((_BLK, _C), lambda i: (i, 0)),
        ],
        out_specs=pl.BlockSpec(memory_space=pltpu.SMEM),
        out_shape=jax.ShapeDtypeStruct((1, 1), jnp.float32),
    )(out, label)
    return total[0, 0] * (1.0 / (_R * _C))
